# Initial kernel scaffold; baseline (speedup 1.0000x reference)
import jax
import jax.numpy as jnp
from jax.experimental import pallas as pl


def kernel(x, edge_index, edge_attr, batch, params):
    raise NotImplementedError("write your pallas kernel here")


# speedup vs baseline: 1.2651x; 1.2651x over previous
import functools

import jax
import jax.numpy as jnp
from jax import lax
from jax.experimental import pallas as pl
from jax.experimental.pallas import tpu as pltpu
from jax.experimental.pallas import tpu_sc as plsc

EMB = 64
RB = 4000
BE = 4096
C = 128


def _encoder(x, W, b):
    n = x.shape[0]
    grid = n // RB
    return pl.pallas_call(
        lambda x_ref, w_ref, b_ref, o_ref: o_ref.__setitem__(
            ..., jnp.dot(x_ref[...], w_ref[...],
                         preferred_element_type=jnp.float32) + b_ref[...]),
        grid=(grid,),
        in_specs=[
            pl.BlockSpec((RB, x.shape[1]), lambda i: (i, 0)),
            pl.BlockSpec(W.shape, lambda i: (0, 0)),
            pl.BlockSpec((1, EMB), lambda i: (0, 0)),
        ],
        out_specs=pl.BlockSpec((RB, EMB), lambda i: (i, 0)),
        out_shape=jax.ShapeDtypeStruct((n, EMB), jnp.float32),
    )(x, W, b.reshape(1, EMB))


def _edge_encoder(edge_attr, W, b, epad):
    grid = epad // BE
    return pl.pallas_call(
        lambda a_ref, w_ref, b_ref, o_ref: o_ref.__setitem__(
            ..., jnp.dot(a_ref[...], w_ref[...],
                         preferred_element_type=jnp.float32) + b_ref[...]),
        grid=(grid,),
        in_specs=[
            pl.BlockSpec((BE, edge_attr.shape[1]), lambda i: (i, 0)),
            pl.BlockSpec(W.shape, lambda i: (0, 0)),
            pl.BlockSpec((1, EMB), lambda i: (0, 0)),
        ],
        out_specs=pl.BlockSpec((BE, EMB), lambda i: (i, 0)),
        out_shape=jax.ShapeDtypeStruct((epad, EMB), jnp.float32),
    )(edge_attr, W, b.reshape(1, EMB))


def _mlp_stats(h, aggp, heps, W1, b1, W2, b2):
    n = h.shape[0]
    grid = n // RB

    def body(heps_ref, h_ref, a0_ref, a1_ref, W1_ref, b1_ref,
             W2_ref, b2_ref, zm_ref, st_ref, acc_ref):
        i = pl.program_id(0)
        agg = (a0_ref[0] + a1_ref[0]).reshape(RB, EMB)
        z1 = heps_ref[0] * h_ref[...] + agg
        t = jnp.maximum(jnp.dot(z1, W1_ref[...],
                                preferred_element_type=jnp.float32)
                        + b1_ref[...], 0.0)
        zm = jnp.dot(t, W2_ref[...],
                     preferred_element_type=jnp.float32) + b2_ref[...]
        zm_ref[...] = zm

        @pl.when(i == 0)
        def _():
            acc_ref[...] = jnp.zeros_like(acc_ref)

        acc_ref[0:1, :] += jnp.sum(zm, axis=0, keepdims=True)
        acc_ref[1:2, :] += jnp.sum(zm * zm, axis=0, keepdims=True)
        st_ref[...] = acc_ref[...]

    zm, st = pl.pallas_call(
        body,
        grid=(grid,),
        in_specs=[
            pl.BlockSpec(memory_space=pltpu.SMEM),
            pl.BlockSpec((RB, EMB), lambda i: (i, 0)),
            pl.BlockSpec((1, RB, EMB), lambda i: (0, i, 0)),
            pl.BlockSpec((1, RB, EMB), lambda i: (1, i, 0)),
            pl.BlockSpec(W1.shape, lambda i: (0, 0)),
            pl.BlockSpec((1, 2 * EMB), lambda i: (0, 0)),
            pl.BlockSpec(W2.shape, lambda i: (0, 0)),
            pl.BlockSpec((1, EMB), lambda i: (0, 0)),
        ],
        out_specs=[
            pl.BlockSpec((RB, EMB), lambda i: (i, 0)),
            pl.BlockSpec((8, EMB), lambda i: (0, 0)),
        ],
        out_shape=[
            jax.ShapeDtypeStruct((n, EMB), jnp.float32),
            jax.ShapeDtypeStruct((8, EMB), jnp.float32),
        ],
        scratch_shapes=[pltpu.VMEM((8, EMB), jnp.float32)],
    )(heps, h, aggp, aggp, W1, b1.reshape(1, 2 * EMB), W2,
      b2.reshape(1, EMB))
    return zm, st


def _bn_res(zm, h, st, g, bb, relu):
    n = zm.shape[0]
    grid = n // RB
    inv_n = 1.0 / n

    def body(zm_ref, h_ref, st_ref, g_ref, b_ref, o_ref):
        mu = st_ref[0:1, :] * inv_n
        var = st_ref[1:2, :] * inv_n - mu * mu
        inv = lax.rsqrt(var + 1e-5)
        y = (zm_ref[...] - mu) * inv * g_ref[...] + b_ref[...]
        if relu:
            y = jnp.maximum(y, 0.0)
        o_ref[...] = y + h_ref[...]

    return pl.pallas_call(
        body,
        grid=(grid,),
        in_specs=[
            pl.BlockSpec((RB, EMB), lambda i: (i, 0)),
            pl.BlockSpec((RB, EMB), lambda i: (i, 0)),
            pl.BlockSpec((8, EMB), lambda i: (0, 0)),
            pl.BlockSpec((1, EMB), lambda i: (0, 0)),
            pl.BlockSpec((1, EMB), lambda i: (0, 0)),
        ],
        out_specs=pl.BlockSpec((RB, EMB), lambda i: (i, 0)),
        out_shape=jax.ShapeDtypeStruct((n, EMB), jnp.float32),
    )(zm, h, st, g.reshape(1, EMB), bb.reshape(1, EMB))


def _bn_res_gate(zm, h, st, g, bb, x, thr, Wp, bp):
    n = zm.shape[0]
    d = x.shape[1]
    grid = n // RB
    inv_n = 1.0 / n

    def body(zm_ref, h_ref, st_ref, g_ref, b_ref, x_ref, t_ref, wp_ref,
             bp_ref, o_ref):
        mu = st_ref[0:1, :] * inv_n
        var = st_ref[1:2, :] * inv_n - mu * mu
        inv = lax.rsqrt(var + 1e-5)
        hfin = (zm_ref[...] - mu) * inv * g_ref[...] + b_ref[...] + h_ref[...]
        v = jnp.dot(hfin, wp_ref[...],
                    preferred_element_type=jnp.float32) + bp_ref[...]
        p = jax.nn.sigmoid(v)
        gate = jnp.where(p > t_ref[...], 1.0, 0.0)
        o_ref[...] = x_ref[...] * gate

    return pl.pallas_call(
        body,
        grid=(grid,),
        in_specs=[
            pl.BlockSpec((RB, EMB), lambda i: (i, 0)),
            pl.BlockSpec((RB, EMB), lambda i: (i, 0)),
            pl.BlockSpec((8, EMB), lambda i: (0, 0)),
            pl.BlockSpec((1, EMB), lambda i: (0, 0)),
            pl.BlockSpec((1, EMB), lambda i: (0, 0)),
            pl.BlockSpec((RB, d), lambda i: (i, 0)),
            pl.BlockSpec((RB, d), lambda i: (i, 0)),
            pl.BlockSpec(Wp.shape, lambda i: (0, 0)),
            pl.BlockSpec((1, d), lambda i: (0, 0)),
        ],
        out_specs=pl.BlockSpec((RB, d), lambda i: (i, 0)),
        out_shape=jax.ShapeDtypeStruct((n, d), jnp.float32),
    )(zm, h, st, g.reshape(1, EMB), bb.reshape(1, EMB), x, thr, Wp,
      bp.reshape(1, d))


def _sc_segment(h2, e2, srcp, dstp, n):
    epad = srcp.shape[0]
    nacc = ((n + 16 + 127) // 128) * 128
    ehalf = epad // 2
    etile = ehalf // 16
    nch = etile // C
    zrows = 512
    zpt = nacc // 16
    mesh = plsc.VectorSubcoreMesh(core_axis_name="c", subcore_axis_name="s")

    @functools.partial(
        pl.kernel,
        out_type=jax.ShapeDtypeStruct((2, 4, nacc, 16), jnp.float32),
        mesh=mesh,
        compiler_params=pltpu.CompilerParams(use_tc_tiling_on_sc=False),
        scratch_types=[
            pltpu.VMEM((C,), jnp.int32),
            pltpu.VMEM((C,), jnp.int32),
            pltpu.VMEM((C,), jnp.int32),
            pltpu.VMEM((C,), jnp.int32),
            pltpu.VMEM((C, 16), jnp.float32),
            pltpu.VMEM((C, 16), jnp.float32),
            pltpu.VMEM((zrows, 16), jnp.float32),
            pltpu.VMEM_SHARED((nacc, 16), jnp.float32),
            pltpu.SemaphoreType.DMA,
            pltpu.SemaphoreType.DMA,
        ],
    )
    def body(h2_hbm, e2_hbm, src_hbm, dst_hbm, out_hbm,
             srcb, dstb, hidx, eidx, hbuf, ebuf, zbuf, acc, sem1, sem2):
        c = lax.axis_index("c")
        s = lax.axis_index("s")

        def zb(i, carry):
            zbuf[i] = jnp.zeros((16,), jnp.float32)
            return carry
        lax.fori_loop(0, zrows, zb, 0)

        lanes = lax.iota(jnp.int32, 16)
        for q in range(4):
            zlo = s * zpt
            nfull = zpt // zrows
            for k in range(nfull):
                pltpu.sync_copy(zbuf, acc.at[pl.ds(zlo + k * zrows, zrows)])
            rem = zpt - nfull * zrows
            if rem:
                pltpu.sync_copy(zbuf.at[pl.ds(0, rem)],
                                acc.at[pl.ds(zlo + nfull * zrows, rem)])
            plsc.subcore_barrier()

            base0 = c * ehalf + s * etile

            def chunk(ci, carry):
                base = base0 + ci * C
                pltpu.sync_copy(src_hbm.at[pl.ds(base, C)], srcb)
                pltpu.sync_copy(dst_hbm.at[pl.ds(base, C)], dstb)
                for j in range(C // 16):
                    sl = pl.ds(j * 16, 16)
                    hidx[sl] = srcb[sl] * 4 + q
                    eidx[sl] = (base + j * 16 + lanes) * 4 + q
                cp1 = pltpu.async_copy(h2_hbm.at[hidx], hbuf, sem1)
                cp2 = pltpu.async_copy(e2_hbm.at[eidx], ebuf, sem2)
                cp1.wait()
                cp2.wait()

                def rowfn(i, cc):
                    hbuf[i] = jnp.maximum(hbuf[i] + ebuf[i], 0.0)
                    return cc
                lax.fori_loop(0, C, rowfn, 0)
                pltpu.sync_copy(hbuf, acc.at[dstb], add=True)
                return carry

            lax.fori_loop(0, nch, chunk, 0)
            plsc.subcore_barrier()
            flo = s * zpt
            pltpu.sync_copy(acc.at[pl.ds(flo, zpt)],
                            out_hbm.at[c, q, pl.ds(flo, zpt)])
            plsc.subcore_barrier()

    return body(h2, e2, srcp, dstp)


def kernel(x, edge_index, edge_attr, batch, params):
    n = x.shape[0]
    e = edge_index.shape[1]
    epad = ((e + 4095) // 4096) * 4096
    pad = epad - e

    srcp = jnp.concatenate(
        [edge_index[0], jnp.zeros((pad,), jnp.int32)]) if pad else edge_index[0]
    dstp = jnp.concatenate(
        [edge_index[1], jnp.full((pad,), n, jnp.int32)]) if pad else edge_index[1]

    h = _encoder(x, params['W_enc'], params['b_enc'])

    for l in range(2):
        p = params['layers'][l]
        enc = _edge_encoder(edge_attr, p['We'], p['be'], epad)
        aggp = _sc_segment(h.reshape(4 * n, 16), enc.reshape(4 * epad, 16),
                           srcp, dstp, n)
        aggp = aggp[:, :, :n].transpose(0, 2, 1, 3).reshape(2, n, EMB)
        heps = (1.0 + p['eps']).reshape(1)
        zm, st = _mlp_stats(h, aggp, heps, p['W1'], p['b1'], p['W2'], p['b2'])
        if l == 0:
            h = _bn_res(zm, h, st, p['bn_g'], p['bn_b'], relu=True)
        else:
            g = jax.random.gumbel(jax.random.key(42), (n, x.shape[1], 2),
                                  jnp.float32)
            thr = 0.5 * (1.0 + g[..., 0] - g[..., 1])
            x2 = _bn_res_gate(zm, h, st, p['bn_g'], p['bn_b'], x, thr,
                              params['Wp'], params['bp'])

    return (x2, edge_index, edge_attr, batch)


# speedup vs baseline: 1.3704x; 1.0833x over previous
import functools

import jax
import jax.numpy as jnp
from jax import lax
from jax.experimental import pallas as pl
from jax.experimental.pallas import tpu as pltpu
from jax.experimental.pallas import tpu_sc as plsc

EMB = 64
RB = 4000
BE = 4096
C = 128
S = 256


def _encoder(x, W, b):
    n = x.shape[0]
    grid = n // RB
    return pl.pallas_call(
        lambda x_ref, w_ref, b_ref, o_ref: o_ref.__setitem__(
            ..., jnp.dot(x_ref[...], w_ref[...],
                         preferred_element_type=jnp.float32) + b_ref[...]),
        grid=(grid,),
        in_specs=[
            pl.BlockSpec((RB, x.shape[1]), lambda i: (i, 0)),
            pl.BlockSpec(W.shape, lambda i: (0, 0)),
            pl.BlockSpec((1, EMB), lambda i: (0, 0)),
        ],
        out_specs=pl.BlockSpec((RB, EMB), lambda i: (i, 0)),
        out_shape=jax.ShapeDtypeStruct((n, EMB), jnp.float32),
    )(x, W, b.reshape(1, EMB))


def _edge_encoder(edge_attr, W, b, epad):
    grid = epad // BE
    return pl.pallas_call(
        lambda a_ref, w_ref, b_ref, o_ref: o_ref.__setitem__(
            ..., jnp.dot(a_ref[...], w_ref[...],
                         preferred_element_type=jnp.float32) + b_ref[...]),
        grid=(grid,),
        in_specs=[
            pl.BlockSpec((BE, edge_attr.shape[1]), lambda i: (i, 0)),
            pl.BlockSpec(W.shape, lambda i: (0, 0)),
            pl.BlockSpec((1, EMB), lambda i: (0, 0)),
        ],
        out_specs=pl.BlockSpec((BE, EMB), lambda i: (i, 0)),
        out_shape=jax.ShapeDtypeStruct((epad, EMB), jnp.float32),
    )(edge_attr, W, b.reshape(1, EMB))


def _mlp_stats(h, aggp, heps, W1, b1, W2, b2):
    n = h.shape[0]
    grid = n // RB

    def body(heps_ref, h_ref, a0_ref, a1_ref, W1_ref, b1_ref,
             W2_ref, b2_ref, zm_ref, st_ref, acc_ref):
        i = pl.program_id(0)
        agg = (a0_ref[0] + a1_ref[0]).reshape(RB, EMB)
        z1 = heps_ref[0] * h_ref[...] + agg
        t = jnp.maximum(jnp.dot(z1, W1_ref[...],
                                preferred_element_type=jnp.float32)
                        + b1_ref[...], 0.0)
        zm = jnp.dot(t, W2_ref[...],
                     preferred_element_type=jnp.float32) + b2_ref[...]
        zm_ref[...] = zm

        @pl.when(i == 0)
        def _():
            acc_ref[...] = jnp.zeros_like(acc_ref)

        acc_ref[0:1, :] += jnp.sum(zm, axis=0, keepdims=True)
        acc_ref[1:2, :] += jnp.sum(zm * zm, axis=0, keepdims=True)
        st_ref[...] = acc_ref[...]

    zm, st = pl.pallas_call(
        body,
        grid=(grid,),
        in_specs=[
            pl.BlockSpec(memory_space=pltpu.SMEM),
            pl.BlockSpec((RB, EMB), lambda i: (i, 0)),
            pl.BlockSpec((1, RB, EMB), lambda i: (0, i, 0)),
            pl.BlockSpec((1, RB, EMB), lambda i: (1, i, 0)),
            pl.BlockSpec(W1.shape, lambda i: (0, 0)),
            pl.BlockSpec((1, 2 * EMB), lambda i: (0, 0)),
            pl.BlockSpec(W2.shape, lambda i: (0, 0)),
            pl.BlockSpec((1, EMB), lambda i: (0, 0)),
        ],
        out_specs=[
            pl.BlockSpec((RB, EMB), lambda i: (i, 0)),
            pl.BlockSpec((8, EMB), lambda i: (0, 0)),
        ],
        out_shape=[
            jax.ShapeDtypeStruct((n, EMB), jnp.float32),
            jax.ShapeDtypeStruct((8, EMB), jnp.float32),
        ],
        scratch_shapes=[pltpu.VMEM((8, EMB), jnp.float32)],
    )(heps, h, aggp, aggp, W1, b1.reshape(1, 2 * EMB), W2,
      b2.reshape(1, EMB))
    return zm, st


def _bn_res(zm, h, st, g, bb, relu):
    n = zm.shape[0]
    grid = n // RB
    inv_n = 1.0 / n

    def body(zm_ref, h_ref, st_ref, g_ref, b_ref, o_ref):
        mu = st_ref[0:1, :] * inv_n
        var = st_ref[1:2, :] * inv_n - mu * mu
        inv = lax.rsqrt(var + 1e-5)
        y = (zm_ref[...] - mu) * inv * g_ref[...] + b_ref[...]
        if relu:
            y = jnp.maximum(y, 0.0)
        o_ref[...] = y + h_ref[...]

    return pl.pallas_call(
        body,
        grid=(grid,),
        in_specs=[
            pl.BlockSpec((RB, EMB), lambda i: (i, 0)),
            pl.BlockSpec((RB, EMB), lambda i: (i, 0)),
            pl.BlockSpec((8, EMB), lambda i: (0, 0)),
            pl.BlockSpec((1, EMB), lambda i: (0, 0)),
            pl.BlockSpec((1, EMB), lambda i: (0, 0)),
        ],
        out_specs=pl.BlockSpec((RB, EMB), lambda i: (i, 0)),
        out_shape=jax.ShapeDtypeStruct((n, EMB), jnp.float32),
    )(zm, h, st, g.reshape(1, EMB), bb.reshape(1, EMB))


def _bn_res_gate(zm, h, st, g, bb, x, thr, Wp, bp):
    n = zm.shape[0]
    d = x.shape[1]
    grid = n // RB
    inv_n = 1.0 / n

    def body(zm_ref, h_ref, st_ref, g_ref, b_ref, x_ref, t_ref, wp_ref,
             bp_ref, o_ref):
        mu = st_ref[0:1, :] * inv_n
        var = st_ref[1:2, :] * inv_n - mu * mu
        inv = lax.rsqrt(var + 1e-5)
        hfin = (zm_ref[...] - mu) * inv * g_ref[...] + b_ref[...] + h_ref[...]
        v = jnp.dot(hfin, wp_ref[...],
                    preferred_element_type=jnp.float32) + bp_ref[...]
        p = jax.nn.sigmoid(v)
        gate = jnp.where(p > t_ref[...], 1.0, 0.0)
        o_ref[...] = x_ref[...] * gate

    return pl.pallas_call(
        body,
        grid=(grid,),
        in_specs=[
            pl.BlockSpec((RB, EMB), lambda i: (i, 0)),
            pl.BlockSpec((RB, EMB), lambda i: (i, 0)),
            pl.BlockSpec((8, EMB), lambda i: (0, 0)),
            pl.BlockSpec((1, EMB), lambda i: (0, 0)),
            pl.BlockSpec((1, EMB), lambda i: (0, 0)),
            pl.BlockSpec((RB, d), lambda i: (i, 0)),
            pl.BlockSpec((RB, d), lambda i: (i, 0)),
            pl.BlockSpec(Wp.shape, lambda i: (0, 0)),
            pl.BlockSpec((1, d), lambda i: (0, 0)),
        ],
        out_specs=pl.BlockSpec((RB, d), lambda i: (i, 0)),
        out_shape=jax.ShapeDtypeStruct((n, d), jnp.float32),
    )(zm, h, st, g.reshape(1, EMB), bb.reshape(1, EMB), x, thr, Wp,
      bp.reshape(1, d))


def _sc_segment(h2, e2, srcp, dstp, n):
    epad = e2.shape[0] // 4
    nacc = ((n + 16 + 127) // 128) * 128
    ehalf = epad // 2
    etile = ehalf // 16
    nch = etile // C
    zrows = 512
    zpt = nacc // 16
    mesh = plsc.VectorSubcoreMesh(core_axis_name="c", subcore_axis_name="s")

    vm = pltpu.VMEM
    @functools.partial(
        pl.kernel,
        out_type=jax.ShapeDtypeStruct((2, 4, nacc, 16), jnp.float32),
        mesh=mesh,
        compiler_params=pltpu.CompilerParams(use_tc_tiling_on_sc=False),
        scratch_types=[
            vm((C,), jnp.int32), vm((C,), jnp.int32),
            vm((C,), jnp.int32), vm((C,), jnp.int32),
            vm((C,), jnp.int32), vm((C,), jnp.int32),
            vm((C,), jnp.int32), vm((C,), jnp.int32),
            vm((C, 16), jnp.float32), vm((C, 16), jnp.float32),
            vm((C, 16), jnp.float32), vm((C, 16), jnp.float32),
            vm((zrows, 16), jnp.float32),
            pltpu.VMEM_SHARED((nacc, 16), jnp.float32),
            pltpu.SemaphoreType.DMA, pltpu.SemaphoreType.DMA,
            pltpu.SemaphoreType.DMA, pltpu.SemaphoreType.DMA,
        ],
    )
    def body(h2_hbm, e2_hbm, src_hbm, dst_hbm, out_hbm,
             src0, src1, dst0, dst1, hix0, hix1, eix0, eix1,
             hb0, hb1, eb0, eb1, zbuf, acc, smh0, smh1, sme0, sme1):
        c = lax.axis_index("c")
        tid = lax.axis_index("s")
        srcb, dstb = (src0, src1), (dst0, dst1)
        hixb, eixb = (hix0, hix1), (eix0, eix1)
        hbuf, ebuf = (hb0, hb1), (eb0, eb1)
        semh, seme = (smh0, smh1), (sme0, sme1)
        base0 = c * ehalf + tid * etile
        lanes = lax.iota(jnp.int32, 16)

        def zb(i, carry):
            zbuf[i] = jnp.zeros((16,), jnp.float32)
            return carry
        lax.fori_loop(0, zrows, zb, 0)

        for q in range(4):
            zlo = tid * zpt
            nfull = zpt // zrows
            for k in range(nfull):
                pltpu.sync_copy(zbuf, acc.at[pl.ds(zlo + k * zrows, zrows)])
            rem = zpt - nfull * zrows
            if rem:
                pltpu.sync_copy(zbuf.at[pl.ds(0, rem)],
                                acc.at[pl.ds(zlo + nfull * zrows, rem)])
            plsc.subcore_barrier()

            def load_chunk(ch, b):
                base = base0 + ch * C
                pltpu.sync_copy(src_hbm.at[pl.ds(base, C)], srcb[b])
                pltpu.sync_copy(dst_hbm.at[pl.ds(base, C)], dstb[b])
                for v in range(C // 16):
                    sl = pl.ds(v * 16, 16)
                    hixb[b][sl] = srcb[b][sl] * 4 + q
                    eixb[b][sl] = (base + v * 16 + lanes) * 4 + q

            def fire_chunk(b):
                return (pltpu.async_copy(h2_hbm.at[hixb[b]], hbuf[b],
                                         semh[b]),
                        pltpu.async_copy(e2_hbm.at[eixb[b]], ebuf[b],
                                         seme[b]))

            def consume_chunk(b):
                def rowfn(i, cc, b=b):
                    hbuf[b][i] = jnp.maximum(hbuf[b][i] + ebuf[b][i], 0.0)
                    return cc
                lax.fori_loop(0, C, rowfn, 0)
                pltpu.sync_copy(hbuf[b], acc.at[dstb[b]], add=True)

            def step(k, carry):
                for b in (0, 1):
                    load_chunk(2 * k + b, b)
                cps = [fire_chunk(0), fire_chunk(1)]
                for b in (0, 1):
                    cps[b][0].wait()
                    cps[b][1].wait()
                for b in (0, 1):
                    consume_chunk(b)
                return carry

            lax.fori_loop(0, nch // 2, step, 0)
            if nch % 2:
                load_chunk(nch - 1, 0)
                cp = fire_chunk(0)
                cp[0].wait()
                cp[1].wait()
                consume_chunk(0)
            plsc.subcore_barrier()
            flo = tid * zpt
            pltpu.sync_copy(acc.at[pl.ds(flo, zpt)],
                            out_hbm.at[c, q, pl.ds(flo, zpt)])
            plsc.subcore_barrier()

    return body(h2, e2, srcp, dstp)


def kernel(x, edge_index, edge_attr, batch, params):
    n = x.shape[0]
    e = edge_index.shape[1]
    epad = ((e + 4095) // 4096) * 4096
    pad = epad - e

    srcp = jnp.concatenate(
        [edge_index[0], jnp.zeros((pad,), jnp.int32)]) if pad else edge_index[0]
    trash = n + (jnp.arange(pad, dtype=jnp.int32) % 64)
    dstp = jnp.concatenate(
        [edge_index[1], trash]) if pad else edge_index[1]

    h = _encoder(x, params['W_enc'], params['b_enc'])

    for l in range(2):
        p = params['layers'][l]
        enc = _edge_encoder(edge_attr, p['We'], p['be'], epad)
        aggp = _sc_segment(h.reshape(4 * n, 16), enc.reshape(4 * epad, 16),
                           srcp, dstp, n)
        aggp = aggp[:, :, :n].transpose(0, 2, 1, 3).reshape(2, n, EMB)
        heps = (1.0 + p['eps']).reshape(1)
        zm, st = _mlp_stats(h, aggp, heps, p['W1'], p['b1'], p['W2'], p['b2'])
        if l == 0:
            h = _bn_res(zm, h, st, p['bn_g'], p['bn_b'], relu=True)
        else:
            g = jax.random.gumbel(jax.random.key(42), (n, x.shape[1], 2),
                                  jnp.float32)
            thr = 0.5 * (1.0 + g[..., 0] - g[..., 1])
            x2 = _bn_res_gate(zm, h, st, p['bn_g'], p['bn_b'], x, thr,
                              params['Wp'], params['bp'])

    return (x2, edge_index, edge_attr, batch)


# speedup vs baseline: 1.8464x; 1.3473x over previous
import functools

import jax
import jax.numpy as jnp
from jax import lax
from jax.experimental import pallas as pl
from jax.experimental.pallas import tpu as pltpu
from jax.experimental.pallas import tpu_sc as plsc

EMB = 64
RB = 4000
BE = 4096
C = 128
S = 256


def _encoder(x, W, b):
    n = x.shape[0]
    grid = n // RB
    return pl.pallas_call(
        lambda x_ref, w_ref, b_ref, o_ref: o_ref.__setitem__(
            ..., jnp.dot(x_ref[...], w_ref[...],
                         preferred_element_type=jnp.float32) + b_ref[...]),
        grid=(grid,),
        in_specs=[
            pl.BlockSpec((RB, x.shape[1]), lambda i: (i, 0)),
            pl.BlockSpec(W.shape, lambda i: (0, 0)),
            pl.BlockSpec((1, EMB), lambda i: (0, 0)),
        ],
        out_specs=pl.BlockSpec((RB, EMB), lambda i: (i, 0)),
        out_shape=jax.ShapeDtypeStruct((n, EMB), jnp.float32),
    )(x, W, b.reshape(1, EMB))


def _edge_encoder(edge_attr, W, b, epad):
    grid = epad // BE
    return pl.pallas_call(
        lambda a_ref, w_ref, b_ref, o_ref: o_ref.__setitem__(
            ..., jnp.dot(a_ref[...], w_ref[...],
                         preferred_element_type=jnp.float32) + b_ref[...]),
        grid=(grid,),
        in_specs=[
            pl.BlockSpec((BE, edge_attr.shape[1]), lambda i: (i, 0)),
            pl.BlockSpec(W.shape, lambda i: (0, 0)),
            pl.BlockSpec((1, EMB), lambda i: (0, 0)),
        ],
        out_specs=pl.BlockSpec((BE, EMB), lambda i: (i, 0)),
        out_shape=jax.ShapeDtypeStruct((epad, EMB), jnp.float32),
    )(edge_attr, W, b.reshape(1, EMB))


def _mlp_stats(h, aggp, heps, W1, b1, W2, b2):
    n = h.shape[0]
    grid = n // RB

    def body(heps_ref, h_ref, a0_ref, a1_ref, W1_ref, b1_ref,
             W2_ref, b2_ref, zm_ref, st_ref, acc_ref):
        i = pl.program_id(0)
        agg = (a0_ref[0] + a1_ref[0]).reshape(RB, EMB)
        z1 = heps_ref[0] * h_ref[...] + agg
        t = jnp.maximum(jnp.dot(z1, W1_ref[...],
                                preferred_element_type=jnp.float32)
                        + b1_ref[...], 0.0)
        zm = jnp.dot(t, W2_ref[...],
                     preferred_element_type=jnp.float32) + b2_ref[...]
        zm_ref[...] = zm

        @pl.when(i == 0)
        def _():
            acc_ref[...] = jnp.zeros_like(acc_ref)

        acc_ref[0:1, :] += jnp.sum(zm, axis=0, keepdims=True)
        acc_ref[1:2, :] += jnp.sum(zm * zm, axis=0, keepdims=True)
        st_ref[...] = acc_ref[...]

    zm, st = pl.pallas_call(
        body,
        grid=(grid,),
        in_specs=[
            pl.BlockSpec(memory_space=pltpu.SMEM),
            pl.BlockSpec((RB, EMB), lambda i: (i, 0)),
            pl.BlockSpec((1, RB, EMB), lambda i: (0, i, 0)),
            pl.BlockSpec((1, RB, EMB), lambda i: (1, i, 0)),
            pl.BlockSpec(W1.shape, lambda i: (0, 0)),
            pl.BlockSpec((1, 2 * EMB), lambda i: (0, 0)),
            pl.BlockSpec(W2.shape, lambda i: (0, 0)),
            pl.BlockSpec((1, EMB), lambda i: (0, 0)),
        ],
        out_specs=[
            pl.BlockSpec((RB, EMB), lambda i: (i, 0)),
            pl.BlockSpec((8, EMB), lambda i: (0, 0)),
        ],
        out_shape=[
            jax.ShapeDtypeStruct((n, EMB), jnp.float32),
            jax.ShapeDtypeStruct((8, EMB), jnp.float32),
        ],
        scratch_shapes=[pltpu.VMEM((8, EMB), jnp.float32)],
    )(heps, h, aggp, aggp, W1, b1.reshape(1, 2 * EMB), W2,
      b2.reshape(1, EMB))
    return zm, st


def _bn_res(zm, h, st, g, bb, relu):
    n = zm.shape[0]
    grid = n // RB
    inv_n = 1.0 / n

    def body(zm_ref, h_ref, st_ref, g_ref, b_ref, o_ref):
        mu = st_ref[0:1, :] * inv_n
        var = st_ref[1:2, :] * inv_n - mu * mu
        inv = lax.rsqrt(var + 1e-5)
        y = (zm_ref[...] - mu) * inv * g_ref[...] + b_ref[...]
        if relu:
            y = jnp.maximum(y, 0.0)
        o_ref[...] = y + h_ref[...]

    return pl.pallas_call(
        body,
        grid=(grid,),
        in_specs=[
            pl.BlockSpec((RB, EMB), lambda i: (i, 0)),
            pl.BlockSpec((RB, EMB), lambda i: (i, 0)),
            pl.BlockSpec((8, EMB), lambda i: (0, 0)),
            pl.BlockSpec((1, EMB), lambda i: (0, 0)),
            pl.BlockSpec((1, EMB), lambda i: (0, 0)),
        ],
        out_specs=pl.BlockSpec((RB, EMB), lambda i: (i, 0)),
        out_shape=jax.ShapeDtypeStruct((n, EMB), jnp.float32),
    )(zm, h, st, g.reshape(1, EMB), bb.reshape(1, EMB))


def _bn_res_gate(zm, h, st, g, bb, x, thr, Wp, bp):
    n = zm.shape[0]
    d = x.shape[1]
    grid = n // RB
    inv_n = 1.0 / n

    def body(zm_ref, h_ref, st_ref, g_ref, b_ref, x_ref, t_ref, wp_ref,
             bp_ref, o_ref):
        mu = st_ref[0:1, :] * inv_n
        var = st_ref[1:2, :] * inv_n - mu * mu
        inv = lax.rsqrt(var + 1e-5)
        hfin = (zm_ref[...] - mu) * inv * g_ref[...] + b_ref[...] + h_ref[...]
        v = jnp.dot(hfin, wp_ref[...],
                    preferred_element_type=jnp.float32) + bp_ref[...]
        p = jax.nn.sigmoid(v)
        gate = jnp.where(p > t_ref[...], 1.0, 0.0)
        o_ref[...] = x_ref[...] * gate

    return pl.pallas_call(
        body,
        grid=(grid,),
        in_specs=[
            pl.BlockSpec((RB, EMB), lambda i: (i, 0)),
            pl.BlockSpec((RB, EMB), lambda i: (i, 0)),
            pl.BlockSpec((8, EMB), lambda i: (0, 0)),
            pl.BlockSpec((1, EMB), lambda i: (0, 0)),
            pl.BlockSpec((1, EMB), lambda i: (0, 0)),
            pl.BlockSpec((RB, d), lambda i: (i, 0)),
            pl.BlockSpec((RB, d), lambda i: (i, 0)),
            pl.BlockSpec(Wp.shape, lambda i: (0, 0)),
            pl.BlockSpec((1, d), lambda i: (0, 0)),
        ],
        out_specs=pl.BlockSpec((RB, d), lambda i: (i, 0)),
        out_shape=jax.ShapeDtypeStruct((n, d), jnp.float32),
    )(zm, h, st, g.reshape(1, EMB), bb.reshape(1, EMB), x, thr, Wp,
      bp.reshape(1, d))


def _sc_segment(h2, e2, srcp, dstp, n):
    epad = e2.shape[0] // 4
    nacc = ((n + 16 + 127) // 128) * 128
    ehalf = epad // 2
    etile = ehalf // 16
    nch = etile // C
    zrows = 128
    zpt = nacc // 16
    mesh = plsc.VectorSubcoreMesh(core_axis_name="c", subcore_axis_name="s")

    vm = pltpu.VMEM
    @functools.partial(
        pl.kernel,
        out_type=jax.ShapeDtypeStruct((2, 4, nacc, 16), jnp.float32),
        mesh=mesh,
        compiler_params=pltpu.CompilerParams(use_tc_tiling_on_sc=False),
        scratch_types=(
            [vm((C,), jnp.int32) for _ in range(4)] +
            [vm((C,), jnp.int32) for _ in range(4)] +
            [vm((C,), jnp.int32) for _ in range(4)] +
            [vm((C,), jnp.int32) for _ in range(4)] +
            [vm((C, 16), jnp.float32) for _ in range(4)] +
            [vm((C, 16), jnp.float32) for _ in range(4)] +
            [vm((zrows, 16), jnp.float32),
             pltpu.VMEM_SHARED((nacc, 16), jnp.float32),
             pltpu.SemaphoreType.DMA, pltpu.SemaphoreType.DMA,
             pltpu.SemaphoreType.DMA]
        ),
    )
    def body(h2_hbm, e2_hbm, src_hbm, dst_hbm, out_hbm, *rest):
        srcb, dstb = rest[0:4], rest[4:8]
        hixb, eixb = rest[8:12], rest[12:16]
        hbuf, ebuf = rest[16:20], rest[20:24]
        zbuf, acc, semi, semh, seme = rest[24:29]
        c = lax.axis_index("c")
        tid = lax.axis_index("s")
        base0 = c * ehalf + tid * etile
        lanes = lax.iota(jnp.int32, 16)

        def zb(i, carry):
            zbuf[i] = jnp.zeros((16,), jnp.float32)
            return carry
        lax.fori_loop(0, zrows, zb, 0)

        for q in range(4):
            zlo = tid * zpt
            nfull = zpt // zrows
            for k in range(nfull):
                pltpu.sync_copy(zbuf, acc.at[pl.ds(zlo + k * zrows, zrows)])
            rem = zpt - nfull * zrows
            if rem:
                pltpu.sync_copy(zbuf.at[pl.ds(0, rem)],
                                acc.at[pl.ds(zlo + nfull * zrows, rem)])
            plsc.subcore_barrier()

            def do_chunks(ch0, nb):
                icps = []
                for b in range(nb):
                    base = base0 + (ch0 + b) * C
                    icps.append(
                        (pltpu.async_copy(src_hbm.at[pl.ds(base, C)],
                                          srcb[b], semi),
                         pltpu.async_copy(dst_hbm.at[pl.ds(base, C)],
                                          dstb[b], semi)))
                for b in range(nb):
                    icps[b][0].wait()
                    icps[b][1].wait()
                cps = []
                for b in range(nb):
                    base = base0 + (ch0 + b) * C
                    for v in range(C // 16):
                        sl = pl.ds(v * 16, 16)
                        hixb[b][sl] = srcb[b][sl] * 4 + q
                        eixb[b][sl] = (base + v * 16 + lanes) * 4 + q
                    cps.append(
                        (pltpu.async_copy(h2_hbm.at[hixb[b]], hbuf[b], semh),
                         pltpu.async_copy(e2_hbm.at[eixb[b]], ebuf[b], seme)))
                for b in range(nb):
                    cps[b][0].wait()
                    cps[b][1].wait()

                    def rowfn(i, cc, b=b):
                        hbuf[b][i] = jnp.maximum(hbuf[b][i] + ebuf[b][i], 0.0)
                        return cc
                    lax.fori_loop(0, C, rowfn, 0)
                    pltpu.sync_copy(hbuf[b], acc.at[dstb[b]], add=True)

            def step(k, carry):
                do_chunks(4 * k, 4)
                return carry

            lax.fori_loop(0, nch // 4, step, 0)
            for r in range(nch % 4):
                do_chunks((nch // 4) * 4 + r, 1)
            plsc.subcore_barrier()
            flo = tid * zpt
            pltpu.sync_copy(acc.at[pl.ds(flo, zpt)],
                            out_hbm.at[c, q, pl.ds(flo, zpt)])
            plsc.subcore_barrier()

    return body(h2, e2, srcp, dstp)


def kernel(x, edge_index, edge_attr, batch, params):
    n = x.shape[0]
    e = edge_index.shape[1]
    epad = ((e + 4095) // 4096) * 4096
    pad = epad - e

    srcp = jnp.concatenate(
        [edge_index[0], jnp.zeros((pad,), jnp.int32)]) if pad else edge_index[0]
    trash = n + (jnp.arange(pad, dtype=jnp.int32) % 64)
    dstp = jnp.concatenate(
        [edge_index[1], trash]) if pad else edge_index[1]

    h = _encoder(x, params['W_enc'], params['b_enc'])

    for l in range(2):
        p = params['layers'][l]
        enc = _edge_encoder(edge_attr, p['We'], p['be'], epad)
        aggp = _sc_segment(h.reshape(4 * n, 16), enc.reshape(4 * epad, 16),
                           srcp, dstp, n)
        aggp = aggp[:, :, :n].transpose(0, 2, 1, 3).reshape(2, n, EMB)
        heps = (1.0 + p['eps']).reshape(1)
        zm, st = _mlp_stats(h, aggp, heps, p['W1'], p['b1'], p['W2'], p['b2'])
        if l == 0:
            h = _bn_res(zm, h, st, p['bn_g'], p['bn_b'], relu=True)
        else:
            g = jax.random.gumbel(jax.random.key(42), (n, x.shape[1], 2),
                                  jnp.float32)
            thr = 0.5 * (1.0 + g[..., 0] - g[..., 1])
            x2 = _bn_res_gate(zm, h, st, p['bn_g'], p['bn_b'], x, thr,
                              params['Wp'], params['bp'])

    return (x2, edge_index, edge_attr, batch)


# speedup vs baseline: 1.9108x; 1.0349x over previous
import functools

import jax
import jax.numpy as jnp
from jax import lax
from jax.experimental import pallas as pl
from jax.experimental.pallas import tpu as pltpu
from jax.experimental.pallas import tpu_sc as plsc

EMB = 64
RB = 4000
BE = 4096
C = 128
S = 256


def _encoder(x, W, b):
    n = x.shape[0]
    grid = n // RB
    return pl.pallas_call(
        lambda x_ref, w_ref, b_ref, o_ref: o_ref.__setitem__(
            ..., jnp.dot(x_ref[...], w_ref[...],
                         preferred_element_type=jnp.float32) + b_ref[...]),
        grid=(grid,),
        in_specs=[
            pl.BlockSpec((RB, x.shape[1]), lambda i: (i, 0)),
            pl.BlockSpec(W.shape, lambda i: (0, 0)),
            pl.BlockSpec((1, EMB), lambda i: (0, 0)),
        ],
        out_specs=pl.BlockSpec((RB, EMB), lambda i: (i, 0)),
        out_shape=jax.ShapeDtypeStruct((n, EMB), jnp.float32),
    )(x, W, b.reshape(1, EMB))


def _edge_encoder(edge_attr, W, b, epad):
    grid = epad // BE
    return pl.pallas_call(
        lambda a_ref, w_ref, b_ref, o_ref: o_ref.__setitem__(
            ..., jnp.dot(a_ref[...], w_ref[...],
                         preferred_element_type=jnp.float32) + b_ref[...]),
        grid=(grid,),
        in_specs=[
            pl.BlockSpec((BE, edge_attr.shape[1]), lambda i: (i, 0)),
            pl.BlockSpec(W.shape, lambda i: (0, 0)),
            pl.BlockSpec((1, EMB), lambda i: (0, 0)),
        ],
        out_specs=pl.BlockSpec((BE, EMB), lambda i: (i, 0)),
        out_shape=jax.ShapeDtypeStruct((epad, EMB), jnp.float32),
    )(edge_attr, W, b.reshape(1, EMB))


def _mlp_stats(h, aggp, heps, W1, b1, W2, b2):
    n = h.shape[0]
    grid = n // RB

    def body(heps_ref, h_ref, a0_ref, a1_ref, W1_ref, b1_ref,
             W2_ref, b2_ref, zm_ref, st_ref, acc_ref):
        i = pl.program_id(0)
        agg = (a0_ref[0] + a1_ref[0]).reshape(RB, EMB)
        z1 = heps_ref[0] * h_ref[...] + agg
        t = jnp.maximum(jnp.dot(z1, W1_ref[...],
                                preferred_element_type=jnp.float32)
                        + b1_ref[...], 0.0)
        zm = jnp.dot(t, W2_ref[...],
                     preferred_element_type=jnp.float32) + b2_ref[...]
        zm_ref[...] = zm

        @pl.when(i == 0)
        def _():
            acc_ref[...] = jnp.zeros_like(acc_ref)

        acc_ref[0:1, :] += jnp.sum(zm, axis=0, keepdims=True)
        acc_ref[1:2, :] += jnp.sum(zm * zm, axis=0, keepdims=True)
        st_ref[...] = acc_ref[...]

    zm, st = pl.pallas_call(
        body,
        grid=(grid,),
        in_specs=[
            pl.BlockSpec(memory_space=pltpu.SMEM),
            pl.BlockSpec((RB, EMB), lambda i: (i, 0)),
            pl.BlockSpec((1, RB, EMB), lambda i: (0, i, 0)),
            pl.BlockSpec((1, RB, EMB), lambda i: (1, i, 0)),
            pl.BlockSpec(W1.shape, lambda i: (0, 0)),
            pl.BlockSpec((1, 2 * EMB), lambda i: (0, 0)),
            pl.BlockSpec(W2.shape, lambda i: (0, 0)),
            pl.BlockSpec((1, EMB), lambda i: (0, 0)),
        ],
        out_specs=[
            pl.BlockSpec((RB, EMB), lambda i: (i, 0)),
            pl.BlockSpec((8, EMB), lambda i: (0, 0)),
        ],
        out_shape=[
            jax.ShapeDtypeStruct((n, EMB), jnp.float32),
            jax.ShapeDtypeStruct((8, EMB), jnp.float32),
        ],
        scratch_shapes=[pltpu.VMEM((8, EMB), jnp.float32)],
    )(heps, h, aggp, aggp, W1, b1.reshape(1, 2 * EMB), W2,
      b2.reshape(1, EMB))
    return zm, st


def _bn_res(zm, h, st, g, bb, relu):
    n = zm.shape[0]
    grid = n // RB
    inv_n = 1.0 / n

    def body(zm_ref, h_ref, st_ref, g_ref, b_ref, o_ref):
        mu = st_ref[0:1, :] * inv_n
        var = st_ref[1:2, :] * inv_n - mu * mu
        inv = lax.rsqrt(var + 1e-5)
        y = (zm_ref[...] - mu) * inv * g_ref[...] + b_ref[...]
        if relu:
            y = jnp.maximum(y, 0.0)
        o_ref[...] = y + h_ref[...]

    return pl.pallas_call(
        body,
        grid=(grid,),
        in_specs=[
            pl.BlockSpec((RB, EMB), lambda i: (i, 0)),
            pl.BlockSpec((RB, EMB), lambda i: (i, 0)),
            pl.BlockSpec((8, EMB), lambda i: (0, 0)),
            pl.BlockSpec((1, EMB), lambda i: (0, 0)),
            pl.BlockSpec((1, EMB), lambda i: (0, 0)),
        ],
        out_specs=pl.BlockSpec((RB, EMB), lambda i: (i, 0)),
        out_shape=jax.ShapeDtypeStruct((n, EMB), jnp.float32),
    )(zm, h, st, g.reshape(1, EMB), bb.reshape(1, EMB))


def _bn_res_gate(zm, h, st, g, bb, x, thr, Wp, bp):
    n = zm.shape[0]
    d = x.shape[1]
    grid = n // RB
    inv_n = 1.0 / n

    def body(zm_ref, h_ref, st_ref, g_ref, b_ref, x_ref, t_ref, wp_ref,
             bp_ref, o_ref):
        mu = st_ref[0:1, :] * inv_n
        var = st_ref[1:2, :] * inv_n - mu * mu
        inv = lax.rsqrt(var + 1e-5)
        hfin = (zm_ref[...] - mu) * inv * g_ref[...] + b_ref[...] + h_ref[...]
        v = jnp.dot(hfin, wp_ref[...],
                    preferred_element_type=jnp.float32) + bp_ref[...]
        p = jax.nn.sigmoid(v)
        gate = jnp.where(p > t_ref[...], 1.0, 0.0)
        o_ref[...] = x_ref[...] * gate

    return pl.pallas_call(
        body,
        grid=(grid,),
        in_specs=[
            pl.BlockSpec((RB, EMB), lambda i: (i, 0)),
            pl.BlockSpec((RB, EMB), lambda i: (i, 0)),
            pl.BlockSpec((8, EMB), lambda i: (0, 0)),
            pl.BlockSpec((1, EMB), lambda i: (0, 0)),
            pl.BlockSpec((1, EMB), lambda i: (0, 0)),
            pl.BlockSpec((RB, d), lambda i: (i, 0)),
            pl.BlockSpec((RB, d), lambda i: (i, 0)),
            pl.BlockSpec(Wp.shape, lambda i: (0, 0)),
            pl.BlockSpec((1, d), lambda i: (0, 0)),
        ],
        out_specs=pl.BlockSpec((RB, d), lambda i: (i, 0)),
        out_shape=jax.ShapeDtypeStruct((n, d), jnp.float32),
    )(zm, h, st, g.reshape(1, EMB), bb.reshape(1, EMB), x, thr, Wp,
      bp.reshape(1, d))


def _sc_segment(h2, e2, srcp, dstp, n):
    epad = e2.shape[0] // 4
    nacc = ((n + 16 + 127) // 128) * 128
    ehalf = epad // 2
    etile = ehalf // 16
    nch = etile // C
    zrows = 64
    zpt = nacc // 16
    mesh = plsc.VectorSubcoreMesh(core_axis_name="c", subcore_axis_name="s")

    vm = pltpu.VMEM
    @functools.partial(
        pl.kernel,
        out_type=jax.ShapeDtypeStruct((2, 4, nacc, 16), jnp.float32),
        mesh=mesh,
        compiler_params=pltpu.CompilerParams(use_tc_tiling_on_sc=False),
        scratch_types=(
            [vm((C,), jnp.int32) for _ in range(6)] +
            [vm((C,), jnp.int32) for _ in range(6)] +
            [vm((C,), jnp.int32) for _ in range(6)] +
            [vm((C,), jnp.int32) for _ in range(6)] +
            [vm((C, 16), jnp.float32) for _ in range(6)] +
            [vm((C, 16), jnp.float32) for _ in range(6)] +
            [vm((zrows, 16), jnp.float32),
             pltpu.VMEM_SHARED((nacc, 16), jnp.float32),
             pltpu.SemaphoreType.DMA, pltpu.SemaphoreType.DMA,
             pltpu.SemaphoreType.DMA]
        ),
    )
    def body(h2_hbm, e2_hbm, src_hbm, dst_hbm, out_hbm, *rest):
        srcb, dstb = rest[0:6], rest[6:12]
        hixb, eixb = rest[12:18], rest[18:24]
        hbuf, ebuf = rest[24:30], rest[30:36]
        zbuf, acc, semi, semh, seme = rest[36:41]
        c = lax.axis_index("c")
        tid = lax.axis_index("s")
        base0 = c * ehalf + tid * etile
        lanes = lax.iota(jnp.int32, 16)

        def zb(i, carry):
            zbuf[i] = jnp.zeros((16,), jnp.float32)
            return carry
        lax.fori_loop(0, zrows, zb, 0)

        for q in range(4):
            zlo = tid * zpt
            nfull = zpt // zrows
            for k in range(nfull):
                pltpu.sync_copy(zbuf, acc.at[pl.ds(zlo + k * zrows, zrows)])
            rem = zpt - nfull * zrows
            if rem:
                pltpu.sync_copy(zbuf.at[pl.ds(0, rem)],
                                acc.at[pl.ds(zlo + nfull * zrows, rem)])
            plsc.subcore_barrier()

            def do_chunks(ch0, nb):
                icps = []
                for b in range(nb):
                    base = base0 + (ch0 + b) * C
                    icps.append(
                        (pltpu.async_copy(src_hbm.at[pl.ds(base, C)],
                                          srcb[b], semi),
                         pltpu.async_copy(dst_hbm.at[pl.ds(base, C)],
                                          dstb[b], semi)))
                for b in range(nb):
                    icps[b][0].wait()
                    icps[b][1].wait()
                cps = []
                for b in range(nb):
                    base = base0 + (ch0 + b) * C
                    for v in range(C // 16):
                        sl = pl.ds(v * 16, 16)
                        hixb[b][sl] = srcb[b][sl] * 4 + q
                        eixb[b][sl] = (base + v * 16 + lanes) * 4 + q
                    cps.append(
                        (pltpu.async_copy(h2_hbm.at[hixb[b]], hbuf[b], semh),
                         pltpu.async_copy(e2_hbm.at[eixb[b]], ebuf[b], seme)))
                for b in range(nb):
                    cps[b][0].wait()
                    cps[b][1].wait()

                    def rowfn(i, cc, b=b):
                        hbuf[b][i] = jnp.maximum(hbuf[b][i] + ebuf[b][i], 0.0)
                        return cc
                    lax.fori_loop(0, C, rowfn, 0)
                    pltpu.sync_copy(hbuf[b], acc.at[dstb[b]], add=True)

            def step(k, carry):
                do_chunks(6 * k, 6)
                return carry

            lax.fori_loop(0, nch // 6, step, 0)
            for r in range(nch % 6):
                do_chunks((nch // 6) * 6 + r, 1)
            plsc.subcore_barrier()
            flo = tid * zpt
            pltpu.sync_copy(acc.at[pl.ds(flo, zpt)],
                            out_hbm.at[c, q, pl.ds(flo, zpt)])
            plsc.subcore_barrier()

    return body(h2, e2, srcp, dstp)


def kernel(x, edge_index, edge_attr, batch, params):
    n = x.shape[0]
    e = edge_index.shape[1]
    epad = ((e + 4095) // 4096) * 4096
    pad = epad - e

    srcp = jnp.concatenate(
        [edge_index[0], jnp.zeros((pad,), jnp.int32)]) if pad else edge_index[0]
    trash = n + (jnp.arange(pad, dtype=jnp.int32) % 64)
    dstp = jnp.concatenate(
        [edge_index[1], trash]) if pad else edge_index[1]

    h = _encoder(x, params['W_enc'], params['b_enc'])

    for l in range(2):
        p = params['layers'][l]
        enc = _edge_encoder(edge_attr, p['We'], p['be'], epad)
        aggp = _sc_segment(h.reshape(4 * n, 16), enc.reshape(4 * epad, 16),
                           srcp, dstp, n)
        aggp = aggp[:, :, :n].transpose(0, 2, 1, 3).reshape(2, n, EMB)
        heps = (1.0 + p['eps']).reshape(1)
        zm, st = _mlp_stats(h, aggp, heps, p['W1'], p['b1'], p['W2'], p['b2'])
        if l == 0:
            h = _bn_res(zm, h, st, p['bn_g'], p['bn_b'], relu=True)
        else:
            g = jax.random.gumbel(jax.random.key(42), (n, x.shape[1], 2),
                                  jnp.float32)
            thr = 0.5 * (1.0 + g[..., 0] - g[..., 1])
            x2 = _bn_res_gate(zm, h, st, p['bn_g'], p['bn_b'], x, thr,
                              params['Wp'], params['bp'])

    return (x2, edge_index, edge_attr, batch)


# speedup vs baseline: 1.9111x; 1.0001x over previous
import functools

import jax
import jax.numpy as jnp
from jax import lax
from jax.experimental import pallas as pl
from jax.experimental.pallas import tpu as pltpu
from jax.experimental.pallas import tpu_sc as plsc

EMB = 64
RB = 4000
BE = 4096
C = 128


def _encoder(x, W, b):
    n = x.shape[0]
    grid = n // RB
    return pl.pallas_call(
        lambda x_ref, w_ref, b_ref, o_ref: o_ref.__setitem__(
            ..., jnp.dot(x_ref[...], w_ref[...],
                         preferred_element_type=jnp.float32) + b_ref[...]),
        grid=(grid,),
        in_specs=[
            pl.BlockSpec((RB, x.shape[1]), lambda i: (i, 0)),
            pl.BlockSpec(W.shape, lambda i: (0, 0)),
            pl.BlockSpec((1, EMB), lambda i: (0, 0)),
        ],
        out_specs=pl.BlockSpec((RB, EMB), lambda i: (i, 0)),
        out_shape=jax.ShapeDtypeStruct((n, EMB), jnp.float32),
    )(x, W, b.reshape(1, EMB))


def _edge_encoder(edge_attr, W, b, epad):
    grid = epad // BE
    return pl.pallas_call(
        lambda a_ref, w_ref, b_ref, o_ref: o_ref.__setitem__(
            ..., jnp.dot(a_ref[...], w_ref[...],
                         preferred_element_type=jnp.float32) + b_ref[...]),
        grid=(grid,),
        in_specs=[
            pl.BlockSpec((BE, edge_attr.shape[1]), lambda i: (i, 0)),
            pl.BlockSpec(W.shape, lambda i: (0, 0)),
            pl.BlockSpec((1, EMB), lambda i: (0, 0)),
        ],
        out_specs=pl.BlockSpec((BE, EMB), lambda i: (i, 0)),
        out_shape=jax.ShapeDtypeStruct((epad, EMB), jnp.float32),
    )(edge_attr, W, b.reshape(1, EMB))


def _mlp_stats(h, aggp, heps, W1, b1, W2, b2):
    n = h.shape[0]
    grid = n // RB

    def body(heps_ref, h_ref, a0_ref, a1_ref, W1_ref, b1_ref,
             W2_ref, b2_ref, zm_ref, st_ref, acc_ref):
        i = pl.program_id(0)
        agg = (a0_ref[0] + a1_ref[0]).reshape(RB, EMB)
        z1 = heps_ref[0] * h_ref[...] + agg
        t = jnp.maximum(jnp.dot(z1, W1_ref[...],
                                preferred_element_type=jnp.float32)
                        + b1_ref[...], 0.0)
        zm = jnp.dot(t, W2_ref[...],
                     preferred_element_type=jnp.float32) + b2_ref[...]
        zm_ref[...] = zm

        @pl.when(i == 0)
        def _():
            acc_ref[...] = jnp.zeros_like(acc_ref)

        acc_ref[0:1, :] += jnp.sum(zm, axis=0, keepdims=True)
        acc_ref[1:2, :] += jnp.sum(zm * zm, axis=0, keepdims=True)
        st_ref[...] = acc_ref[...]

    zm, st = pl.pallas_call(
        body,
        grid=(grid,),
        in_specs=[
            pl.BlockSpec(memory_space=pltpu.SMEM),
            pl.BlockSpec((RB, EMB), lambda i: (i, 0)),
            pl.BlockSpec((1, RB, EMB), lambda i: (0, i, 0)),
            pl.BlockSpec((1, RB, EMB), lambda i: (1, i, 0)),
            pl.BlockSpec(W1.shape, lambda i: (0, 0)),
            pl.BlockSpec((1, 2 * EMB), lambda i: (0, 0)),
            pl.BlockSpec(W2.shape, lambda i: (0, 0)),
            pl.BlockSpec((1, EMB), lambda i: (0, 0)),
        ],
        out_specs=[
            pl.BlockSpec((RB, EMB), lambda i: (i, 0)),
            pl.BlockSpec((8, EMB), lambda i: (0, 0)),
        ],
        out_shape=[
            jax.ShapeDtypeStruct((n, EMB), jnp.float32),
            jax.ShapeDtypeStruct((8, EMB), jnp.float32),
        ],
        scratch_shapes=[pltpu.VMEM((8, EMB), jnp.float32)],
    )(heps, h, aggp, aggp, W1, b1.reshape(1, 2 * EMB), W2,
      b2.reshape(1, EMB))
    return zm, st


def _bn_res(zm, h, st, g, bb, relu):
    n = zm.shape[0]
    grid = n // RB
    inv_n = 1.0 / n

    def body(zm_ref, h_ref, st_ref, g_ref, b_ref, o_ref):
        mu = st_ref[0:1, :] * inv_n
        var = st_ref[1:2, :] * inv_n - mu * mu
        inv = lax.rsqrt(var + 1e-5)
        y = (zm_ref[...] - mu) * inv * g_ref[...] + b_ref[...]
        if relu:
            y = jnp.maximum(y, 0.0)
        o_ref[...] = y + h_ref[...]

    return pl.pallas_call(
        body,
        grid=(grid,),
        in_specs=[
            pl.BlockSpec((RB, EMB), lambda i: (i, 0)),
            pl.BlockSpec((RB, EMB), lambda i: (i, 0)),
            pl.BlockSpec((8, EMB), lambda i: (0, 0)),
            pl.BlockSpec((1, EMB), lambda i: (0, 0)),
            pl.BlockSpec((1, EMB), lambda i: (0, 0)),
        ],
        out_specs=pl.BlockSpec((RB, EMB), lambda i: (i, 0)),
        out_shape=jax.ShapeDtypeStruct((n, EMB), jnp.float32),
    )(zm, h, st, g.reshape(1, EMB), bb.reshape(1, EMB))


def _bn_res_gate(zm, h, st, g, bb, x, thr, Wp, bp):
    n = zm.shape[0]
    d = x.shape[1]
    grid = n // RB
    inv_n = 1.0 / n

    def body(zm_ref, h_ref, st_ref, g_ref, b_ref, x_ref, t_ref, wp_ref,
             bp_ref, o_ref):
        mu = st_ref[0:1, :] * inv_n
        var = st_ref[1:2, :] * inv_n - mu * mu
        inv = lax.rsqrt(var + 1e-5)
        hfin = (zm_ref[...] - mu) * inv * g_ref[...] + b_ref[...] + h_ref[...]
        v = jnp.dot(hfin, wp_ref[...],
                    preferred_element_type=jnp.float32) + bp_ref[...]
        p = jax.nn.sigmoid(v)
        gate = jnp.where(p > t_ref[...], 1.0, 0.0)
        o_ref[...] = x_ref[...] * gate

    return pl.pallas_call(
        body,
        grid=(grid,),
        in_specs=[
            pl.BlockSpec((RB, EMB), lambda i: (i, 0)),
            pl.BlockSpec((RB, EMB), lambda i: (i, 0)),
            pl.BlockSpec((8, EMB), lambda i: (0, 0)),
            pl.BlockSpec((1, EMB), lambda i: (0, 0)),
            pl.BlockSpec((1, EMB), lambda i: (0, 0)),
            pl.BlockSpec((RB, d), lambda i: (i, 0)),
            pl.BlockSpec((RB, d), lambda i: (i, 0)),
            pl.BlockSpec(Wp.shape, lambda i: (0, 0)),
            pl.BlockSpec((1, d), lambda i: (0, 0)),
        ],
        out_specs=pl.BlockSpec((RB, d), lambda i: (i, 0)),
        out_shape=jax.ShapeDtypeStruct((n, d), jnp.float32),
    )(zm, h, st, g.reshape(1, EMB), bb.reshape(1, EMB), x, thr, Wp,
      bp.reshape(1, d))


def _sc_segment(h2, e2, srcp, dstp, n):
    epad = e2.shape[0] // 4
    nacc = ((n + 16 + 127) // 128) * 128
    ehalf = epad // 2
    etile = ehalf // 16
    nch = etile // C
    zrows = 64
    zpt = nacc // 16
    mesh = plsc.VectorSubcoreMesh(core_axis_name="c", subcore_axis_name="s")

    vm = pltpu.VMEM
    @functools.partial(
        pl.kernel,
        out_type=jax.ShapeDtypeStruct((2, 4, nacc, 16), jnp.float32),
        mesh=mesh,
        compiler_params=pltpu.CompilerParams(use_tc_tiling_on_sc=False),
        scratch_types=(
            [vm((C,), jnp.int32) for _ in range(6)] +
            [vm((C,), jnp.int32) for _ in range(6)] +
            [vm((C,), jnp.int32) for _ in range(6)] +
            [vm((C,), jnp.int32) for _ in range(6)] +
            [vm((C, 16), jnp.float32) for _ in range(6)] +
            [vm((C, 16), jnp.float32) for _ in range(6)] +
            [vm((zrows, 16), jnp.float32),
             pltpu.VMEM_SHARED((nacc, 16), jnp.float32),
             pltpu.SemaphoreType.DMA, pltpu.SemaphoreType.DMA,
             pltpu.SemaphoreType.DMA]
        ),
    )
    def body(h2_hbm, e2_hbm, src_hbm, dst_hbm, out_hbm, *rest):
        srcb, dstb = rest[0:6], rest[6:12]
        hixb, eixb = rest[12:18], rest[18:24]
        hbuf, ebuf = rest[24:30], rest[30:36]
        zbuf, acc, semi, semh, seme = rest[36:41]
        c = lax.axis_index("c")
        tid = lax.axis_index("s")
        base0 = c * ehalf + tid * etile
        lanes = lax.iota(jnp.int32, 16)

        def zb(i, carry):
            zbuf[i] = jnp.zeros((16,), jnp.float32)
            return carry
        lax.fori_loop(0, zrows, zb, 0)

        for q in range(4):
            zlo = tid * zpt
            nfull = zpt // zrows
            for k in range(nfull):
                pltpu.sync_copy(zbuf, acc.at[pl.ds(zlo + k * zrows, zrows)])
            rem = zpt - nfull * zrows
            if rem:
                pltpu.sync_copy(zbuf.at[pl.ds(0, rem)],
                                acc.at[pl.ds(zlo + nfull * zrows, rem)])
            plsc.subcore_barrier()

            def do_chunks(ch0, nb):
                icps = []
                for b in range(nb):
                    base = base0 + (ch0 + b) * C
                    icps.append(
                        (pltpu.async_copy(src_hbm.at[pl.ds(base, C)],
                                          srcb[b], semi),
                         pltpu.async_copy(dst_hbm.at[pl.ds(base, C)],
                                          dstb[b], semi)))
                for b in range(nb):
                    icps[b][0].wait()
                    icps[b][1].wait()
                cps = []
                for b in range(nb):
                    base = base0 + (ch0 + b) * C
                    for v in range(C // 16):
                        sl = pl.ds(v * 16, 16)
                        hixb[b][sl] = srcb[b][sl] * 4 + q
                        eixb[b][sl] = (base + v * 16 + lanes) * 4 + q
                    cps.append(
                        (pltpu.async_copy(h2_hbm.at[hixb[b]], hbuf[b], semh),
                         pltpu.async_copy(e2_hbm.at[eixb[b]], ebuf[b], seme)))
                for b in range(nb):
                    cps[b][0].wait()
                    cps[b][1].wait()

                    def rowfn(i, cc, b=b):
                        hbuf[b][i] = jnp.maximum(hbuf[b][i] + ebuf[b][i], 0.0)
                        return cc
                    lax.fori_loop(0, C, rowfn, 0)
                    pltpu.sync_copy(hbuf[b], acc.at[dstb[b]], add=True)

            def step(k, carry):
                do_chunks(6 * k, 6)
                return carry

            lax.fori_loop(0, nch // 6, step, 0)
            for r in range(nch % 6):
                do_chunks((nch // 6) * 6 + r, 1)
            plsc.subcore_barrier()
            flo = tid * zpt
            pltpu.sync_copy(acc.at[pl.ds(flo, zpt)],
                            out_hbm.at[c, q, pl.ds(flo, zpt)])
            plsc.subcore_barrier()

    return body(h2, e2, srcp, dstp)


def kernel(x, edge_index, edge_attr, batch, params):
    n = x.shape[0]
    e = edge_index.shape[1]
    epad = ((e + 4095) // 4096) * 4096
    pad = epad - e

    srcp = jnp.concatenate(
        [edge_index[0], jnp.zeros((pad,), jnp.int32)]) if pad else edge_index[0]
    trash = n + (jnp.arange(pad, dtype=jnp.int32) % 64)
    dstp = jnp.concatenate(
        [edge_index[1], trash]) if pad else edge_index[1]

    h = _encoder(x, params['W_enc'], params['b_enc'])

    for l in range(2):
        p = params['layers'][l]
        enc = _edge_encoder(edge_attr, p['We'], p['be'], epad)
        aggp = _sc_segment(h.reshape(4 * n, 16), enc.reshape(4 * epad, 16),
                           srcp, dstp, n)
        aggp = aggp[:, :, :n].transpose(0, 2, 1, 3).reshape(2, n, EMB)
        heps = (1.0 + p['eps']).reshape(1)
        zm, st = _mlp_stats(h, aggp, heps, p['W1'], p['b1'], p['W2'], p['b2'])
        if l == 0:
            h = _bn_res(zm, h, st, p['bn_g'], p['bn_b'], relu=True)
        else:
            g = jax.random.gumbel(jax.random.key(42), (n, x.shape[1], 2),
                                  jnp.float32)
            thr = 0.5 * (1.0 + g[..., 0] - g[..., 1])
            x2 = _bn_res_gate(zm, h, st, p['bn_g'], p['bn_b'], x, thr,
                              params['Wp'], params['bp'])

    return (x2, edge_index, edge_attr, batch)
